# baseline (speedup 1.0000x reference)
import functools

import jax
import jax.numpy as jnp
from jax import lax
from jax.experimental import pallas as pl
from jax.experimental.pallas import tpu as pltpu
from jax.experimental.pallas import tpu_sc as plsc

T = 128
NC, NS = 2, 16
NW = NC * NS
CHUNK = 48
L = 16


def _row_indices(tile_e_v, counts_v, offs_v, ntoffs_v, order_v, h, S, B):
    r = h & (T - 1)
    bb = (h >> (T.bit_length() - 1)) & (B - 1)
    t = h >> ((B * T).bit_length() - 1)
    e = plsc.load_gather(tile_e_v, [t])
    cnt = plsc.load_gather(counts_v, [e])
    off = plsc.load_gather(offs_v, [e])
    nt_off = plsc.load_gather(ntoffs_v, [e])
    jr = (t - nt_off) * T + r
    valid = jr < cnt
    p = jnp.minimum(off + jr, S - 1)
    tok = plsc.load_gather(order_v, [p])
    return tok + bb * S, valid


def _sc_gather(x2, order, tile_e32, counts, offs, ntoffs, ucvec, G, NT2):
    R, D = x2.shape
    S = order.shape[0]
    B = R // S
    per_w = G // NW
    n_chunks = per_w // CHUNK
    assert per_w * NW == G and n_chunks * CHUNK == per_w

    mesh = plsc.VectorSubcoreMesh(core_axis_name="c", subcore_axis_name="s")

    @functools.partial(
        pl.kernel,
        out_type=jax.ShapeDtypeStruct((G, D), jnp.float32),
        mesh=mesh,
        compiler_params=pltpu.CompilerParams(needs_layout_passes=False),
        scratch_types=[
            pltpu.VMEM((S,), jnp.int32),
            pltpu.VMEM((NT2,), jnp.int32),
            pltpu.VMEM((L,), jnp.int32),
            pltpu.VMEM((L,), jnp.int32),
            pltpu.VMEM((L,), jnp.int32),
            pltpu.VMEM((L,), jnp.int32),
            pltpu.VMEM((n_chunks, CHUNK), jnp.int32),
            pltpu.VMEM((2, CHUNK, D), jnp.float32),
            pltpu.SemaphoreType.DMA,
            pltpu.SemaphoreType.DMA,
        ],
    )
    def k(x_hbm, order_hbm, tile_e_hbm, counts_hbm, offs_hbm, ntoffs_hbm,
          uc_hbm, out_hbm, order_v, tile_e_v, counts_v, offs_v, ntoffs_v,
          uc_v, idx_v, rows_v, gsem, osem):
        wid = lax.axis_index("s") * NC + lax.axis_index("c")
        pltpu.sync_copy(order_hbm, order_v)
        pltpu.sync_copy(tile_e_hbm, tile_e_v)
        pltpu.sync_copy(counts_hbm, counts_v)
        pltpu.sync_copy(offs_hbm, offs_v)
        pltpu.sync_copy(ntoffs_hbm, ntoffs_v)
        pltpu.sync_copy(uc_hbm, uc_v)
        uc = jnp.max(uc_v[...])
        per_c = CHUNK // L
        for v in range(per_w // L):
            c, g = v // per_c, v % per_c
            h = (wid + NW * c) * CHUNK + g * L + lax.iota(jnp.int32, L)
            gidx, _ = _row_indices(tile_e_v, counts_v, offs_v, ntoffs_v, order_v, h, S, B)
            idx_v[c, pl.ds(g * L, L)] = gidx

        def in_copy(c):
            return pltpu.make_async_copy(x_hbm.at[idx_v.at[c]],
                                         rows_v.at[c % 2], gsem)

        def out_copy(c):
            off = (wid + NW * c) * CHUNK
            return pltpu.make_async_copy(rows_v.at[c % 2],
                                         out_hbm.at[pl.ds(off, CHUNK)], osem)

        def act(c):
            return (wid + NW * c) < uc

        for c in range(n_chunks):
            if c >= 2:
                @pl.when(act(c - 2))
                def _wo():
                    out_copy(c - 2).wait()

            @pl.when(act(c))
            def _si():
                in_copy(c).start()
            if c >= 1:
                @pl.when(act(c - 1))
                def _wi():
                    in_copy(c - 1).wait()
                    out_copy(c - 1).start()

        @pl.when(act(n_chunks - 1))
        def _tail1():
            in_copy(n_chunks - 1).wait()
            out_copy(n_chunks - 1).start()
        if n_chunks >= 2:
            @pl.when(act(n_chunks - 2))
            def _tail2():
                out_copy(n_chunks - 2).wait()

        @pl.when(act(n_chunks - 1))
        def _tail3():
            out_copy(n_chunks - 1).wait()

    return k(x2, order, tile_e32, counts, offs, ntoffs, ucvec)


def _sc_out_gather(ys, order, mi, offs, ntoffs, n_out_rows, S, B):
    G, D = ys.shape
    per_w = n_out_rows // NW
    OCH = 32
    n_chunks = per_w // OCH
    assert per_w * NW == n_out_rows and n_chunks * OCH == per_w

    mesh = plsc.VectorSubcoreMesh(core_axis_name="c", subcore_axis_name="s")

    @functools.partial(
        pl.kernel,
        out_type=jax.ShapeDtypeStruct((n_out_rows, D), jnp.float32),
        mesh=mesh,
        compiler_params=pltpu.CompilerParams(needs_layout_passes=False),
        scratch_types=[
            pltpu.VMEM((S,), jnp.int32),
            pltpu.VMEM((S,), jnp.int32),
            pltpu.VMEM((S,), jnp.int32),
            pltpu.VMEM((L,), jnp.int32),
            pltpu.VMEM((L,), jnp.int32),
            pltpu.VMEM((n_chunks, OCH), jnp.int32),
            pltpu.VMEM((2, OCH, D), jnp.float32),
            pltpu.SemaphoreType.DMA,
            pltpu.SemaphoreType.DMA,
        ],
    )
    def k(ys_hbm, order_hbm, mi_hbm, offs_hbm, ntoffs_hbm, out_hbm,
          order_v, mi_v, inv_v, offs_v, ntoffs_v, idx_v, rows_v, isem, osem):
        wid = lax.axis_index("s") * NC + lax.axis_index("c")
        base = wid * per_w
        pltpu.sync_copy(order_hbm, order_v)
        pltpu.sync_copy(mi_hbm, mi_v)
        pltpu.sync_copy(offs_hbm, offs_v)
        pltpu.sync_copy(ntoffs_hbm, ntoffs_v)
        for g in range(S // L):
            pvec = g * L + lax.iota(jnp.int32, L)
            toks = order_v[pl.ds(g * L, L)]
            plsc.store_scatter(inv_v, [toks], pvec)
        sshift = S.bit_length() - 1
        for v in range(per_w // L):
            m = base + v * L + lax.iota(jnp.int32, L)
            i = m & (S - 1)
            b = m >> sshift
            e = plsc.load_gather(mi_v, [i])
            rank = plsc.load_gather(inv_v, [i])
            q = rank - plsc.load_gather(offs_v, [e])
            tile = plsc.load_gather(ntoffs_v, [e]) + (q >> (T.bit_length() - 1))
            hsrc = tile * (B * T) + b * T + (q & (T - 1))
            idx_v[v // (OCH // L), pl.ds((v % (OCH // L)) * L, L)] = hsrc

        def start_in(c):
            return pltpu.async_copy(ys_hbm.at[idx_v.at[c]],
                                    rows_v.at[c % 2], isem)

        def start_out(c):
            off = base + c * OCH
            return pltpu.async_copy(rows_v.at[c % 2],
                                    out_hbm.at[pl.ds(off, OCH)], osem)

        copies_in = [None] * n_chunks
        copies_out = [None] * n_chunks
        for c in range(n_chunks):
            if c >= 2:
                copies_out[c - 2].wait()
            copies_in[c] = start_in(c)
            if c >= 1:
                copies_in[c - 1].wait()
                copies_out[c - 1] = start_out(c - 1)
        copies_in[n_chunks - 1].wait()
        copies_out[n_chunks - 1] = start_out(n_chunks - 1)
        if n_chunks >= 2:
            copies_out[n_chunks - 2].wait()
        copies_out[n_chunks - 1].wait()

    return k(ys, order, mi, offs, ntoffs)


def _tc_grouped_matmul(xs, W, b3, meta2, n_tiles, rows_per_tile):
    D = xs.shape[1]
    Dout = W.shape[2]
    R = rows_per_tile

    KS = 2

    def body(m_ref, xs_ref, w_ref, b_ref, o_ref):
        del m_ref
        acc = jnp.dot(xs_ref[...].astype(jnp.bfloat16), w_ref[0],
                      preferred_element_type=jnp.float32)

        @pl.when(pl.program_id(1) == 0)
        def _first():
            o_ref[...] = acc + b_ref[0]

        @pl.when(pl.program_id(1) != 0)
        def _rest():
            o_ref[...] += acc

    grid_spec = pltpu.PrefetchScalarGridSpec(
        num_scalar_prefetch=1,
        grid=(n_tiles, KS),
        in_specs=[
            pl.BlockSpec((R, D // KS), lambda t, k, m: (m[0, t], k)),
            pl.BlockSpec((1, D // KS, Dout), lambda t, k, m: (m[1, t], k, 0)),
            pl.BlockSpec((1, 1, Dout), lambda t, k, m: (m[1, t], 0, 0)),
        ],
        out_specs=pl.BlockSpec((R, Dout), lambda t, k, m: (m[0, t], 0)),
    )
    return pl.pallas_call(
        body,
        grid_spec=grid_spec,
        out_shape=jax.ShapeDtypeStruct((n_tiles * R, Dout), jnp.float32),
    )(meta2, xs, W, b3)


def kernel(x, module_indices, W, b):
    B, S, D = x.shape
    E, _, Dout = W.shape
    NT = S // T + E
    R = B * T
    G = NT * R

    mi = module_indices.astype(jnp.int32)
    order = jnp.argsort(mi).astype(jnp.int32)
    counts = jnp.sum((mi[None, :] == jnp.arange(E, dtype=jnp.int32)[:, None])
                     .astype(jnp.int32), axis=1)
    ccum = jnp.cumsum(counts)
    offs = ccum - counts
    nt = (counts + T - 1) // T
    ntcum = jnp.cumsum(nt)
    nt_offs = ntcum - nt
    tclamp = jnp.minimum(jnp.arange(NT, dtype=jnp.int32), ntcum[-1] - 1)
    tile_e = jnp.minimum(
        jnp.sum((ntcum[None, :] <= tclamp[:, None]).astype(jnp.int32), axis=1),
        E - 1).astype(jnp.int32)

    NT2 = 2 * L
    tile_e32 = jnp.concatenate(
        [tile_e, jnp.full((NT2 - NT,), E - 1, jnp.int32)])
    pad = jnp.zeros((L - E,), jnp.int32)
    counts16 = jnp.concatenate([counts.astype(jnp.int32), pad])
    offs16 = jnp.concatenate([offs.astype(jnp.int32), pad])
    ntoffs16 = jnp.concatenate([nt_offs.astype(jnp.int32), pad])

    used_rows = ntcum[-1] * R
    uc = (used_rows + CHUNK - 1) // CHUNK
    uc16 = jnp.full((L,), uc, jnp.int32)
    x2 = x.reshape(B * S, D)
    xs = _sc_gather(x2, order, tile_e32, counts16, offs16, ntoffs16, uc16, G, NT2)
    b3 = b.reshape(E, 1, Dout)
    Wb = W.astype(jnp.bfloat16)
    meta2 = jnp.stack([tclamp, tile_e])
    ys = _tc_grouped_matmul(xs, Wb, b3, meta2, NT, R)
    out_buf = _sc_out_gather(ys, order, mi, offs16, ntoffs16, B * S, S, B)
    return out_buf.reshape(B, S, Dout)


# speedup vs baseline: 1.2283x; 1.2283x over previous
import functools

import jax
import jax.numpy as jnp
from jax import lax
from jax.experimental import pallas as pl
from jax.experimental.pallas import tpu as pltpu
from jax.experimental.pallas import tpu_sc as plsc

T = 128
NC, NS = 2, 16
NW = NC * NS
CHUNK = 48
L = 16


def _row_indices(tile_e_v, counts_v, offs_v, ntoffs_v, order_v, h, S, B):
    r = h & (T - 1)
    bb = (h >> (T.bit_length() - 1)) & (B - 1)
    t = h >> ((B * T).bit_length() - 1)
    e = plsc.load_gather(tile_e_v, [t])
    cnt = plsc.load_gather(counts_v, [e])
    off = plsc.load_gather(offs_v, [e])
    nt_off = plsc.load_gather(ntoffs_v, [e])
    jr = (t - nt_off) * T + r
    valid = jr < cnt
    p = jnp.minimum(off + jr, S - 1)
    tok = plsc.load_gather(order_v, [p])
    return tok + bb * S, valid


def _sc_gather(x2, order, tile_e32, counts, offs, ntoffs, ucvec, G, NT2):
    R, D = x2.shape
    S = order.shape[0]
    B = R // S
    per_w = G // NW
    n_chunks = per_w // CHUNK
    assert per_w * NW == G and n_chunks * CHUNK == per_w

    mesh = plsc.VectorSubcoreMesh(core_axis_name="c", subcore_axis_name="s")

    @functools.partial(
        pl.kernel,
        out_type=jax.ShapeDtypeStruct((G, D), jnp.float32),
        mesh=mesh,
        compiler_params=pltpu.CompilerParams(needs_layout_passes=False),
        scratch_types=[
            pltpu.VMEM((S,), jnp.int32),
            pltpu.VMEM((NT2,), jnp.int32),
            pltpu.VMEM((L,), jnp.int32),
            pltpu.VMEM((L,), jnp.int32),
            pltpu.VMEM((L,), jnp.int32),
            pltpu.VMEM((L,), jnp.int32),
            pltpu.VMEM((n_chunks, CHUNK), jnp.int32),
            pltpu.VMEM((2, CHUNK, D), jnp.float32),
            pltpu.SemaphoreType.DMA,
            pltpu.SemaphoreType.DMA,
        ],
    )
    def k(x_hbm, order_hbm, tile_e_hbm, counts_hbm, offs_hbm, ntoffs_hbm,
          uc_hbm, out_hbm, order_v, tile_e_v, counts_v, offs_v, ntoffs_v,
          uc_v, idx_v, rows_v, gsem, osem):
        wid = lax.axis_index("s") * NC + lax.axis_index("c")
        pltpu.sync_copy(order_hbm, order_v)
        pltpu.sync_copy(tile_e_hbm, tile_e_v)
        pltpu.sync_copy(counts_hbm, counts_v)
        pltpu.sync_copy(offs_hbm, offs_v)
        pltpu.sync_copy(ntoffs_hbm, ntoffs_v)
        pltpu.sync_copy(uc_hbm, uc_v)
        uc = jnp.max(uc_v[...])
        per_c = CHUNK // L
        for v in range(per_w // L):
            c, g = v // per_c, v % per_c
            h = (wid + NW * c) * CHUNK + g * L + lax.iota(jnp.int32, L)
            gidx, _ = _row_indices(tile_e_v, counts_v, offs_v, ntoffs_v, order_v, h, S, B)
            idx_v[c, pl.ds(g * L, L)] = gidx

        def in_copy(c):
            return pltpu.make_async_copy(x_hbm.at[idx_v.at[c]],
                                         rows_v.at[c % 2], gsem)

        def out_copy(c):
            off = (wid + NW * c) * CHUNK
            return pltpu.make_async_copy(rows_v.at[c % 2],
                                         out_hbm.at[pl.ds(off, CHUNK)], osem)

        def act(c):
            return (wid + NW * c) < uc

        for c in range(n_chunks):
            if c >= 2:
                @pl.when(act(c - 2))
                def _wo():
                    out_copy(c - 2).wait()

            @pl.when(act(c))
            def _si():
                in_copy(c).start()
            if c >= 1:
                @pl.when(act(c - 1))
                def _wi():
                    in_copy(c - 1).wait()
                    out_copy(c - 1).start()

        @pl.when(act(n_chunks - 1))
        def _tail1():
            in_copy(n_chunks - 1).wait()
            out_copy(n_chunks - 1).start()
        if n_chunks >= 2:
            @pl.when(act(n_chunks - 2))
            def _tail2():
                out_copy(n_chunks - 2).wait()

        @pl.when(act(n_chunks - 1))
        def _tail3():
            out_copy(n_chunks - 1).wait()

    return k(x2, order, tile_e32, counts, offs, ntoffs, ucvec)


def _sc_out_gather(ys, order, mi, offs, ntoffs, n_out_rows, S, B):
    G, D = ys.shape
    per_w = n_out_rows // NW
    OCH = 32
    n_chunks = per_w // OCH
    assert per_w * NW == n_out_rows and n_chunks * OCH == per_w

    mesh = plsc.VectorSubcoreMesh(core_axis_name="c", subcore_axis_name="s")

    @functools.partial(
        pl.kernel,
        out_type=jax.ShapeDtypeStruct((n_out_rows, D), jnp.float32),
        mesh=mesh,
        compiler_params=pltpu.CompilerParams(needs_layout_passes=False),
        scratch_types=[
            pltpu.VMEM((S,), jnp.int32),
            pltpu.VMEM((S,), jnp.int32),
            pltpu.VMEM((S,), jnp.int32),
            pltpu.VMEM((L,), jnp.int32),
            pltpu.VMEM((L,), jnp.int32),
            pltpu.VMEM((n_chunks, OCH), jnp.int32),
            pltpu.VMEM((2, OCH, D), jnp.float32),
            pltpu.SemaphoreType.DMA,
            pltpu.SemaphoreType.DMA,
        ],
    )
    def k(ys_hbm, order_hbm, mi_hbm, offs_hbm, ntoffs_hbm, out_hbm,
          order_v, mi_v, inv_v, offs_v, ntoffs_v, idx_v, rows_v, isem, osem):
        wid = lax.axis_index("s") * NC + lax.axis_index("c")
        base = wid * per_w
        pltpu.sync_copy(order_hbm, order_v)
        pltpu.sync_copy(mi_hbm, mi_v)
        pltpu.sync_copy(offs_hbm, offs_v)
        pltpu.sync_copy(ntoffs_hbm, ntoffs_v)
        for g in range(S // L):
            pvec = g * L + lax.iota(jnp.int32, L)
            toks = order_v[pl.ds(g * L, L)]
            plsc.store_scatter(inv_v, [toks], pvec)
        sshift = S.bit_length() - 1
        for v in range(per_w // L):
            m = base + v * L + lax.iota(jnp.int32, L)
            i = m & (S - 1)
            b = m >> sshift
            e = plsc.load_gather(mi_v, [i])
            rank = plsc.load_gather(inv_v, [i])
            q = rank - plsc.load_gather(offs_v, [e])
            tile = plsc.load_gather(ntoffs_v, [e]) + (q >> (T.bit_length() - 1))
            hsrc = tile * (B * T) + b * T + (q & (T - 1))
            idx_v[v // (OCH // L), pl.ds((v % (OCH // L)) * L, L)] = hsrc

        def start_in(c):
            return pltpu.async_copy(ys_hbm.at[idx_v.at[c]],
                                    rows_v.at[c % 2], isem)

        def start_out(c):
            off = base + c * OCH
            return pltpu.async_copy(rows_v.at[c % 2],
                                    out_hbm.at[pl.ds(off, OCH)], osem)

        copies_in = [None] * n_chunks
        copies_out = [None] * n_chunks
        for c in range(n_chunks):
            if c >= 2:
                copies_out[c - 2].wait()
            copies_in[c] = start_in(c)
            if c >= 1:
                copies_in[c - 1].wait()
                copies_out[c - 1] = start_out(c - 1)
        copies_in[n_chunks - 1].wait()
        copies_out[n_chunks - 1] = start_out(n_chunks - 1)
        if n_chunks >= 2:
            copies_out[n_chunks - 2].wait()
        copies_out[n_chunks - 1].wait()

    return k(ys, order, mi, offs, ntoffs)


def _tc_grouped_matmul(xs, W, b3, meta2, n_tiles, rows_per_tile):
    D = xs.shape[1]
    Dout = W.shape[2]
    R = rows_per_tile

    def body(m_ref, xs_ref, w_ref, b_ref, o_ref):
        del m_ref
        acc = jnp.dot(xs_ref[...].astype(jnp.bfloat16), w_ref[0],
                      preferred_element_type=jnp.float32)
        o_ref[...] = acc + b_ref[0]

    grid_spec = pltpu.PrefetchScalarGridSpec(
        num_scalar_prefetch=1,
        grid=(n_tiles,),
        in_specs=[
            pl.BlockSpec((R, D), lambda t, m: (m[0, t], 0)),
            pl.BlockSpec((1, D, Dout), lambda t, m: (m[1, t], 0, 0)),
            pl.BlockSpec((1, 1, Dout), lambda t, m: (m[1, t], 0, 0)),
        ],
        out_specs=pl.BlockSpec((R, Dout), lambda t, m: (m[0, t], 0)),
    )
    return pl.pallas_call(
        body,
        grid_spec=grid_spec,
        out_shape=jax.ShapeDtypeStruct((n_tiles * R, Dout), jnp.float32),
    )(meta2, xs, W, b3)


def kernel(x, module_indices, W, b):
    B, S, D = x.shape
    E, _, Dout = W.shape
    NT = S // T + E
    R = B * T
    G = NT * R

    mi = module_indices.astype(jnp.int32)
    order = jnp.argsort(mi).astype(jnp.int32)
    counts = jnp.sum((mi[None, :] == jnp.arange(E, dtype=jnp.int32)[:, None])
                     .astype(jnp.int32), axis=1)
    ccum = jnp.cumsum(counts)
    offs = ccum - counts
    nt = (counts + T - 1) // T
    ntcum = jnp.cumsum(nt)
    nt_offs = ntcum - nt
    tclamp = jnp.minimum(jnp.arange(NT, dtype=jnp.int32), ntcum[-1] - 1)
    tile_e = jnp.minimum(
        jnp.sum((ntcum[None, :] <= tclamp[:, None]).astype(jnp.int32), axis=1),
        E - 1).astype(jnp.int32)

    NT2 = 2 * L
    tile_e32 = jnp.concatenate(
        [tile_e, jnp.full((NT2 - NT,), E - 1, jnp.int32)])
    pad = jnp.zeros((L - E,), jnp.int32)
    counts16 = jnp.concatenate([counts.astype(jnp.int32), pad])
    offs16 = jnp.concatenate([offs.astype(jnp.int32), pad])
    ntoffs16 = jnp.concatenate([nt_offs.astype(jnp.int32), pad])

    used_rows = ntcum[-1] * R
    uc = (used_rows + CHUNK - 1) // CHUNK
    uc16 = jnp.full((L,), uc, jnp.int32)
    x2 = x.reshape(B * S, D)
    xs = _sc_gather(x2, order, tile_e32, counts16, offs16, ntoffs16, uc16, G, NT2)
    b3 = b.reshape(E, 1, Dout)
    Wb = W.astype(jnp.bfloat16)
    meta2 = jnp.stack([tclamp, tile_e])
    ys = _tc_grouped_matmul(xs, Wb, b3, meta2, NT, R)
    out_buf = _sc_out_gather(ys, order, mi, offs16, ntoffs16, B * S, S, B)
    return out_buf.reshape(B, S, Dout)


# speedup vs baseline: 1.2431x; 1.0121x over previous
import functools

import jax
import jax.numpy as jnp
from jax import lax
from jax.experimental import pallas as pl
from jax.experimental.pallas import tpu as pltpu
from jax.experimental.pallas import tpu_sc as plsc

T = 128
NC, NS = 2, 16
NW = NC * NS
CHUNK = 32
NBUF = 3
L = 16


def _row_indices(tile_e_v, counts_v, offs_v, ntoffs_v, order_v, h, S, B):
    r = h & (T - 1)
    bb = (h >> (T.bit_length() - 1)) & (B - 1)
    t = h >> ((B * T).bit_length() - 1)
    e = plsc.load_gather(tile_e_v, [t])
    cnt = plsc.load_gather(counts_v, [e])
    off = plsc.load_gather(offs_v, [e])
    nt_off = plsc.load_gather(ntoffs_v, [e])
    jr = (t - nt_off) * T + r
    valid = jr < cnt
    p = jnp.minimum(off + jr, S - 1)
    tok = plsc.load_gather(order_v, [p])
    return tok + bb * S, valid


def _sc_gather(x2, order, tile_e32, counts, offs, ntoffs, ucvec, G, NT2):
    R, D = x2.shape
    S = order.shape[0]
    B = R // S
    per_w = G // NW
    n_chunks = per_w // CHUNK
    assert per_w * NW == G and n_chunks * CHUNK == per_w

    mesh = plsc.VectorSubcoreMesh(core_axis_name="c", subcore_axis_name="s")

    @functools.partial(
        pl.kernel,
        out_type=jax.ShapeDtypeStruct((G, D), jnp.float32),
        mesh=mesh,
        compiler_params=pltpu.CompilerParams(needs_layout_passes=False),
        scratch_types=[
            pltpu.VMEM((S,), jnp.int32),
            pltpu.VMEM((NT2,), jnp.int32),
            pltpu.VMEM((L,), jnp.int32),
            pltpu.VMEM((L,), jnp.int32),
            pltpu.VMEM((L,), jnp.int32),
            pltpu.VMEM((L,), jnp.int32),
            pltpu.VMEM((n_chunks, CHUNK), jnp.int32),
            pltpu.VMEM((NBUF, CHUNK, D), jnp.float32),
            pltpu.SemaphoreType.DMA,
            pltpu.SemaphoreType.DMA,
        ],
    )
    def k(x_hbm, order_hbm, tile_e_hbm, counts_hbm, offs_hbm, ntoffs_hbm,
          uc_hbm, out_hbm, order_v, tile_e_v, counts_v, offs_v, ntoffs_v,
          uc_v, idx_v, rows_v, gsem, osem):
        wid = lax.axis_index("s") * NC + lax.axis_index("c")
        pltpu.sync_copy(order_hbm, order_v)
        pltpu.sync_copy(tile_e_hbm, tile_e_v)
        pltpu.sync_copy(counts_hbm, counts_v)
        pltpu.sync_copy(offs_hbm, offs_v)
        pltpu.sync_copy(ntoffs_hbm, ntoffs_v)
        pltpu.sync_copy(uc_hbm, uc_v)
        uc = jnp.max(uc_v[...])
        per_c = CHUNK // L
        for v in range(per_w // L):
            c, g = v // per_c, v % per_c
            h = (wid + NW * c) * CHUNK + g * L + lax.iota(jnp.int32, L)
            gidx, _ = _row_indices(tile_e_v, counts_v, offs_v, ntoffs_v, order_v, h, S, B)
            idx_v[c, pl.ds(g * L, L)] = gidx

        def in_copy(c):
            return pltpu.make_async_copy(x_hbm.at[idx_v.at[c]],
                                         rows_v.at[c % NBUF], gsem)

        def out_copy(c):
            off = (wid + NW * c) * CHUNK
            return pltpu.make_async_copy(rows_v.at[c % NBUF],
                                         out_hbm.at[pl.ds(off, CHUNK)], osem)

        def act(c):
            return (wid + NW * c) < uc

        for c in range(n_chunks):
            if c >= NBUF:
                @pl.when(act(c - NBUF))
                def _wo():
                    out_copy(c - NBUF).wait()

            @pl.when(act(c))
            def _si():
                in_copy(c).start()
            if c >= 1:
                @pl.when(act(c - 1))
                def _wi():
                    in_copy(c - 1).wait()
                    out_copy(c - 1).start()

        @pl.when(act(n_chunks - 1))
        def _tail1():
            in_copy(n_chunks - 1).wait()
            out_copy(n_chunks - 1).start()
        for c in range(max(n_chunks - NBUF + 1, 0), n_chunks):
            @pl.when(act(c))
            def _tailo():
                out_copy(c).wait()

    return k(x2, order, tile_e32, counts, offs, ntoffs, ucvec)


def _sc_out_gather(ys, order, mi, offs, ntoffs, n_out_rows, S, B):
    G, D = ys.shape
    per_w = n_out_rows // NW
    OCH = 32
    n_chunks = per_w // OCH
    assert per_w * NW == n_out_rows and n_chunks * OCH == per_w

    mesh = plsc.VectorSubcoreMesh(core_axis_name="c", subcore_axis_name="s")

    @functools.partial(
        pl.kernel,
        out_type=jax.ShapeDtypeStruct((n_out_rows, D), jnp.float32),
        mesh=mesh,
        compiler_params=pltpu.CompilerParams(needs_layout_passes=False),
        scratch_types=[
            pltpu.VMEM((S,), jnp.int32),
            pltpu.VMEM((S,), jnp.int32),
            pltpu.VMEM((S,), jnp.int32),
            pltpu.VMEM((L,), jnp.int32),
            pltpu.VMEM((L,), jnp.int32),
            pltpu.VMEM((n_chunks, OCH), jnp.int32),
            pltpu.VMEM((2, OCH, D), jnp.float32),
            pltpu.SemaphoreType.DMA,
            pltpu.SemaphoreType.DMA,
        ],
    )
    def k(ys_hbm, order_hbm, mi_hbm, offs_hbm, ntoffs_hbm, out_hbm,
          order_v, mi_v, inv_v, offs_v, ntoffs_v, idx_v, rows_v, isem, osem):
        wid = lax.axis_index("s") * NC + lax.axis_index("c")
        base = wid * per_w
        pltpu.sync_copy(order_hbm, order_v)
        pltpu.sync_copy(mi_hbm, mi_v)
        pltpu.sync_copy(offs_hbm, offs_v)
        pltpu.sync_copy(ntoffs_hbm, ntoffs_v)
        for g in range(S // L):
            pvec = g * L + lax.iota(jnp.int32, L)
            toks = order_v[pl.ds(g * L, L)]
            plsc.store_scatter(inv_v, [toks], pvec)
        sshift = S.bit_length() - 1
        for v in range(per_w // L):
            m = base + v * L + lax.iota(jnp.int32, L)
            i = m & (S - 1)
            b = m >> sshift
            e = plsc.load_gather(mi_v, [i])
            rank = plsc.load_gather(inv_v, [i])
            q = rank - plsc.load_gather(offs_v, [e])
            tile = plsc.load_gather(ntoffs_v, [e]) + (q >> (T.bit_length() - 1))
            hsrc = tile * (B * T) + b * T + (q & (T - 1))
            idx_v[v // (OCH // L), pl.ds((v % (OCH // L)) * L, L)] = hsrc

        def start_in(c):
            return pltpu.async_copy(ys_hbm.at[idx_v.at[c]],
                                    rows_v.at[c % 2], isem)

        def start_out(c):
            off = base + c * OCH
            return pltpu.async_copy(rows_v.at[c % 2],
                                    out_hbm.at[pl.ds(off, OCH)], osem)

        copies_in = [None] * n_chunks
        copies_out = [None] * n_chunks
        for c in range(n_chunks):
            if c >= 2:
                copies_out[c - 2].wait()
            copies_in[c] = start_in(c)
            if c >= 1:
                copies_in[c - 1].wait()
                copies_out[c - 1] = start_out(c - 1)
        copies_in[n_chunks - 1].wait()
        copies_out[n_chunks - 1] = start_out(n_chunks - 1)
        if n_chunks >= 2:
            copies_out[n_chunks - 2].wait()
        copies_out[n_chunks - 1].wait()

    return k(ys, order, mi, offs, ntoffs)


def _tc_grouped_matmul(xs, W, b3, meta2, n_tiles, rows_per_tile):
    D = xs.shape[1]
    Dout = W.shape[2]
    R = rows_per_tile

    def body(m_ref, xs_ref, w_ref, b_ref, o_ref):
        del m_ref
        acc = jnp.dot(xs_ref[...].astype(jnp.bfloat16), w_ref[0],
                      preferred_element_type=jnp.float32)
        o_ref[...] = acc + b_ref[0]

    grid_spec = pltpu.PrefetchScalarGridSpec(
        num_scalar_prefetch=1,
        grid=(n_tiles,),
        in_specs=[
            pl.BlockSpec((R, D), lambda t, m: (m[0, t], 0)),
            pl.BlockSpec((1, D, Dout), lambda t, m: (m[1, t], 0, 0)),
            pl.BlockSpec((1, 1, Dout), lambda t, m: (m[1, t], 0, 0)),
        ],
        out_specs=pl.BlockSpec((R, Dout), lambda t, m: (m[0, t], 0)),
    )
    return pl.pallas_call(
        body,
        grid_spec=grid_spec,
        out_shape=jax.ShapeDtypeStruct((n_tiles * R, Dout), jnp.float32),
    )(meta2, xs, W, b3)


def kernel(x, module_indices, W, b):
    B, S, D = x.shape
    E, _, Dout = W.shape
    NT = S // T + E
    R = B * T
    G = NT * R

    mi = module_indices.astype(jnp.int32)
    order = jnp.argsort(mi).astype(jnp.int32)
    counts = jnp.sum((mi[None, :] == jnp.arange(E, dtype=jnp.int32)[:, None])
                     .astype(jnp.int32), axis=1)
    ccum = jnp.cumsum(counts)
    offs = ccum - counts
    nt = (counts + T - 1) // T
    ntcum = jnp.cumsum(nt)
    nt_offs = ntcum - nt
    tclamp = jnp.minimum(jnp.arange(NT, dtype=jnp.int32), ntcum[-1] - 1)
    tile_e = jnp.minimum(
        jnp.sum((ntcum[None, :] <= tclamp[:, None]).astype(jnp.int32), axis=1),
        E - 1).astype(jnp.int32)

    NT2 = 2 * L
    tile_e32 = jnp.concatenate(
        [tile_e, jnp.full((NT2 - NT,), E - 1, jnp.int32)])
    pad = jnp.zeros((L - E,), jnp.int32)
    counts16 = jnp.concatenate([counts.astype(jnp.int32), pad])
    offs16 = jnp.concatenate([offs.astype(jnp.int32), pad])
    ntoffs16 = jnp.concatenate([nt_offs.astype(jnp.int32), pad])

    used_rows = ntcum[-1] * R
    uc = (used_rows + CHUNK - 1) // CHUNK
    uc16 = jnp.full((L,), uc, jnp.int32)
    x2 = x.reshape(B * S, D)
    xs = _sc_gather(x2, order, tile_e32, counts16, offs16, ntoffs16, uc16, G, NT2)
    b3 = b.reshape(E, 1, Dout)
    Wb = W.astype(jnp.bfloat16)
    meta2 = jnp.stack([tclamp, tile_e])
    ys = _tc_grouped_matmul(xs, Wb, b3, meta2, NT, R)
    out_buf = _sc_out_gather(ys, order, mi, offs16, ntoffs16, B * S, S, B)
    return out_buf.reshape(B, S, Dout)
